# baseline (speedup 1.0000x reference)
import numpy as np

import jax
import jax.numpy as jnp
from jax.experimental import pallas as pl
from jax.experimental.pallas import tpu as pltpu
from jax.experimental.shard_map import shard_map
from jax.sharding import Mesh, PartitionSpec as P

_P = 0.2
_THRESH = -(2 ** 31) + int(round(_P * (2 ** 32)))
_SCALE = 1.0 / (1.0 - _P)
_C_COL = -1640531527
_C_ROW = -2049221013
_F = 8


def _srl(v, k):
    return jnp.bitwise_and(jnp.right_shift(v, k), (1 << (32 - k)) - 1)


def _mix(z):
    z = z ^ _srl(z, 16)
    z = z * jnp.int32(0x7FEB352D)
    z = z ^ _srl(z, 15)
    z = z * jnp.int32(-2073777525)
    z = z ^ _srl(z, 16)
    return z


def _mlp_kernel(seed_ref, x_ref, zp_ref, w1_ref, w2_ref, w3_ref, b3_ref,
                o_ref):
    x = x_ref[...]
    t = x.shape[1] // _F
    x_f = jnp.concatenate(
        [x[:, q * t:(q + 1) * t] for q in range(_F)]
        + [jnp.full((_F, t), 1.0, jnp.float32)], axis=0)

    h1 = jnp.maximum(
        jnp.dot(w1_ref[...], x_f, preferred_element_type=jnp.float32), 0.0)
    h2 = jnp.maximum(
        jnp.dot(w2_ref[...], h1, preferred_element_type=jnp.float32), 0.0)

    c_tile = (_C_COL * _F * t + 2 ** 31) % (2 ** 32) - 2 ** 31
    s = seed_ref[0] + pl.program_id(0) * jnp.int32(c_tile)
    z = zp_ref[...] + s
    z = _mix(z)
    h2 = jnp.where(z >= jnp.int32(_THRESH), h2, 0.0)

    out = jnp.dot(w3_ref[...], h2, preferred_element_type=jnp.float32)
    out = out + b3_ref[...]
    o_ref[...] = jnp.concatenate(
        [out[3 * q:3 * q + 3, :] for q in range(_F)], axis=1).astype(o_ref.dtype)


def kernel(x, w1, b1, w2, b2, w3, b3, seed):
    B = x.shape[0]
    tile_b = 131072
    assert B % tile_b == 0
    grid = B // tile_b

    x_t = x.T
    seed_arr = jnp.full((1,), seed, dtype=jnp.int32)

    eye = jnp.eye(_F, dtype=jnp.float32)
    w1_f = jnp.einsum('jf,qp->jqpf', w1, eye).reshape(10 * _F, 4 * _F)
    b1_f = jnp.einsum('j,qp->jqp', b1[:, 0], eye).reshape(10 * _F, _F)
    ones_rows = jnp.concatenate([jnp.zeros((_F, 4 * _F), jnp.float32), eye],
                                axis=1)
    w1_a = jnp.concatenate([jnp.concatenate([w1_f, b1_f], axis=1),
                            ones_rows], axis=0)
    b2_cols = jnp.einsum('k,qp->kqp', b2[:, 0], eye).reshape(10 * _F, _F)
    w2_f = jnp.concatenate([jnp.kron(w2, eye), b2_cols], axis=1)
    w3_f = jnp.einsum('kj,qp->qkjp', w3 * jnp.float32(_SCALE),
                      eye).reshape(3 * _F, 10 * _F)
    b3_f = jnp.tile(b3, (_F, 1))

    t = tile_b // _F
    row = jax.lax.broadcasted_iota(jnp.int32, (10 * _F, t), 0)
    lane = jax.lax.broadcasted_iota(jnp.int32, (10 * _F, t), 1)
    zp = ((jnp.bitwise_and(row, _F - 1) * jnp.int32(t) + lane)
          * jnp.int32(_C_COL)
          + jax.lax.shift_right_logical(row, 3) * jnp.int32(_C_ROW))

    def full(shape):
        return pl.BlockSpec(shape, lambda b, s: (0, 0))

    def run(seed_loc, x_loc, *ws):
        b_loc = x_loc.shape[1]
        return pl.pallas_call(
            _mlp_kernel,
            out_shape=jax.ShapeDtypeStruct((3, b_loc), jnp.float32),
            grid_spec=pltpu.PrefetchScalarGridSpec(
                num_scalar_prefetch=1,
                grid=(b_loc // tile_b,),
                in_specs=[
                    pl.BlockSpec((4, tile_b), lambda b, s: (0, b)),
                    full(zp.shape),
                    full(w1_a.shape),
                    full(w2_f.shape), full(w3_f.shape), full(b3_f.shape),
                ],
                out_specs=pl.BlockSpec((3, tile_b), lambda b, s: (0, b)),
            ),
            compiler_params=pltpu.CompilerParams(
                dimension_semantics=("parallel",),
            ),
        )(seed_loc, x_loc, *ws)

    devs = jax.devices()
    nd = len(devs) if len(devs) > 1 and B % (len(devs) * tile_b) == 0 else 1
    if nd > 1:
        c_shard = (_C_COL * (B // nd) + 2 ** 31) % (2 ** 32) - 2 ** 31

        def run_shard(seed_rep, x_loc, *ws):
            i = jax.lax.axis_index('d')
            seed_loc = seed_rep + jnp.int32(c_shard) * i.astype(jnp.int32)
            return run(seed_loc, x_loc, *ws)

        mesh = Mesh(np.asarray(devs), ('d',))
        out_t = shard_map(
            run_shard, mesh=mesh,
            in_specs=(P(), P(None, 'd'), P(), P(), P(), P(), P()),
            out_specs=P(None, 'd'), check_rep=False,
        )(seed_arr, x_t, zp, w1_a, w2_f, w3_f, b3_f)
    else:
        out_t = run(seed_arr, x_t, zp, w1_a, w2_f, w3_f, b3_f)

    return out_t.T


# speedup vs baseline: 4.0306x; 4.0306x over previous
import jax
import jax.numpy as jnp
from jax.experimental import pallas as pl
from jax.experimental.pallas import tpu as pltpu

_P = 0.2
_THRESH = -(2 ** 31) + int(round(_P * (2 ** 32)))
_SCALE = 1.0 / (1.0 - _P)
_C_COL = -1640531527
_C_ROW = -2049221013
_F = 8


def _srl(v, k):
    return jnp.bitwise_and(jnp.right_shift(v, k), (1 << (32 - k)) - 1)


def _mix(z):
    z = z ^ _srl(z, 16)
    z = z * jnp.int32(0x7FEB352D)
    z = z ^ _srl(z, 15)
    z = z * jnp.int32(-2073777525)
    z = z ^ _srl(z, 16)
    return z


def _mlp_kernel(seed_ref, x_ref, zp_ref, w1_ref, w2_ref, w3_ref, b3_ref,
                o_ref):
    x = x_ref[...]
    t = x.shape[1] // _F
    x_f = jnp.concatenate(
        [x[:, q * t:(q + 1) * t] for q in range(_F)]
        + [jnp.full((_F, t), 1.0, jnp.float32)], axis=0)

    h1 = jnp.maximum(
        jnp.dot(w1_ref[...], x_f, preferred_element_type=jnp.float32), 0.0)
    h2 = jnp.maximum(
        jnp.dot(w2_ref[...], h1, preferred_element_type=jnp.float32), 0.0)

    c_tile = (_C_COL * _F * t + 2 ** 31) % (2 ** 32) - 2 ** 31
    s = seed_ref[0] + pl.program_id(0) * jnp.int32(c_tile)
    z = zp_ref[...] + s
    z = _mix(z)
    h2 = jnp.where(z >= jnp.int32(_THRESH), h2, 0.0)

    out = jnp.dot(w3_ref[...], h2, preferred_element_type=jnp.float32)
    out = out + b3_ref[...]
    o_ref[...] = jnp.concatenate(
        [out[4 * q:4 * q + 3, :] for q in range(_F)], axis=1).astype(o_ref.dtype)


def kernel(x, w1, b1, w2, b2, w3, b3, seed):
    B = x.shape[0]
    tile_b = 131072
    assert B % tile_b == 0
    grid = B // tile_b

    x_t = x.T
    seed_arr = jnp.full((1,), seed, dtype=jnp.int32)

    eye = jnp.eye(_F, dtype=jnp.float32)
    w1_f = jnp.einsum('jf,qp->jqpf', w1, eye).reshape(10 * _F, 4 * _F)
    b1_f = jnp.einsum('j,qp->jqp', b1[:, 0], eye).reshape(10 * _F, _F)
    ones_rows = jnp.concatenate([jnp.zeros((_F, 4 * _F), jnp.float32), eye],
                                axis=1)
    w1_a = jnp.concatenate([jnp.concatenate([w1_f, b1_f], axis=1),
                            ones_rows], axis=0)
    b2_cols = jnp.einsum('k,qp->kqp', b2[:, 0], eye).reshape(10 * _F, _F)
    w2_f = jnp.concatenate([jnp.kron(w2, eye), b2_cols], axis=1)
    w3_p = jnp.concatenate([w3 * jnp.float32(_SCALE),
                            jnp.zeros((1, 10), jnp.float32)], axis=0)
    w3_f = jnp.einsum('kj,qp->qkjp', w3_p, eye).reshape(4 * _F, 10 * _F)
    b3_f = jnp.tile(jnp.concatenate([b3, jnp.zeros((1, 1), jnp.float32)]),
                    (_F, 1))

    t = tile_b // _F
    row = jax.lax.broadcasted_iota(jnp.int32, (10 * _F, t), 0)
    lane = jax.lax.broadcasted_iota(jnp.int32, (10 * _F, t), 1)
    zp = ((jnp.bitwise_and(row, _F - 1) * jnp.int32(t) + lane)
          * jnp.int32(_C_COL)
          + jax.lax.shift_right_logical(row, 3) * jnp.int32(_C_ROW))

    def full(shape):
        return pl.BlockSpec(shape, lambda b, s: (0, 0))

    out_t = pl.pallas_call(
        _mlp_kernel,
        out_shape=jax.ShapeDtypeStruct((3, B), jnp.float32),
        grid_spec=pltpu.PrefetchScalarGridSpec(
            num_scalar_prefetch=1,
            grid=(grid,),
            in_specs=[
                pl.BlockSpec((4, tile_b), lambda b, s: (0, b)),
                full(zp.shape),
                full(w1_a.shape),
                full(w2_f.shape), full(w3_f.shape), full(b3_f.shape),
            ],
            out_specs=pl.BlockSpec((3, tile_b), lambda b, s: (0, b)),
        ),
        compiler_params=pltpu.CompilerParams(
            dimension_semantics=("parallel",),
        ),
    )(seed_arr, x_t, zp, w1_a, w2_f, w3_f, b3_f)

    return out_t.T


# speedup vs baseline: 4.2529x; 1.0552x over previous
import jax
import jax.numpy as jnp
from jax.experimental import pallas as pl
from jax.experimental.pallas import tpu as pltpu

_P = 0.2
_THRESH = -(2 ** 31) + int(round(_P * (2 ** 32)))
_SCALE = 1.0 / (1.0 - _P)
_C_COL = -1640531527
_C_ROW = -2049221013
_F = 8


def _srl(v, k):
    return jnp.bitwise_and(jnp.right_shift(v, k), (1 << (32 - k)) - 1)


def _mix(z):
    z = z ^ _srl(z, 16)
    z = z * jnp.int32(0x7FEB352D)
    z = z ^ _srl(z, 15)
    z = z * jnp.int32(-2073777525)
    z = z ^ _srl(z, 16)
    return z


def _mlp_kernel(seed_ref, x_ref, zp_ref, w1_ref, w2_ref, w3_ref, b3_ref,
                o_ref):
    x = x_ref[...]
    t = x.shape[1] // _F
    x_f = jnp.concatenate(
        [x[:, q * t:(q + 1) * t] for q in range(_F)]
        + [jnp.full((_F, t), 1.0, jnp.float32)], axis=0)

    h1 = jnp.maximum(
        jnp.dot(w1_ref[...], x_f, preferred_element_type=jnp.float32), 0.0)
    h2 = jnp.maximum(
        jnp.dot(w2_ref[...], h1, preferred_element_type=jnp.float32), 0.0)

    c_tile = (_C_COL * _F * t + 2 ** 31) % (2 ** 32) - 2 ** 31
    s = seed_ref[0] + pl.program_id(0) * jnp.int32(c_tile)
    z = zp_ref[...] + s
    z = _mix(z)
    h2 = jnp.where(z >= jnp.int32(_THRESH), h2, 0.0)

    out = jnp.dot(w3_ref[...], h2, preferred_element_type=jnp.float32)
    out = out + b3_ref[...]
    o_ref[...] = jnp.concatenate(
        [out[3 * q:3 * q + 3, :] for q in range(_F)], axis=1).astype(o_ref.dtype)


def kernel(x, w1, b1, w2, b2, w3, b3, seed):
    B = x.shape[0]
    tile_b = 131072
    assert B % tile_b == 0
    grid = B // tile_b

    x_t = x.T
    seed_arr = jnp.full((1,), seed, dtype=jnp.int32)

    eye = jnp.eye(_F, dtype=jnp.float32)
    w1_f = jnp.einsum('jf,qp->jqpf', w1, eye).reshape(10 * _F, 4 * _F)
    b1_f = jnp.einsum('j,qp->jqp', b1[:, 0], eye).reshape(10 * _F, _F)
    ones_rows = jnp.concatenate([jnp.zeros((_F, 4 * _F), jnp.float32), eye],
                                axis=1)
    w1_a = jnp.concatenate([jnp.concatenate([w1_f, b1_f], axis=1),
                            ones_rows], axis=0)
    b2_cols = jnp.einsum('k,qp->kqp', b2[:, 0], eye).reshape(10 * _F, _F)
    w2_f = jnp.concatenate([jnp.kron(w2, eye), b2_cols], axis=1)
    w3_f = jnp.einsum('kj,qp->qkjp', w3 * jnp.float32(_SCALE),
                      eye).reshape(3 * _F, 10 * _F)
    b3_f = jnp.tile(b3, (_F, 1))

    t = tile_b // _F
    row = jax.lax.broadcasted_iota(jnp.int32, (10 * _F, t), 0)
    lane = jax.lax.broadcasted_iota(jnp.int32, (10 * _F, t), 1)
    zp = ((jnp.bitwise_and(row, _F - 1) * jnp.int32(t) + lane)
          * jnp.int32(_C_COL)
          + jax.lax.shift_right_logical(row, 3) * jnp.int32(_C_ROW))

    def full(shape):
        return pl.BlockSpec(shape, lambda b, s: (0, 0))

    out_t = pl.pallas_call(
        _mlp_kernel,
        out_shape=jax.ShapeDtypeStruct((3, B), jnp.float32),
        grid_spec=pltpu.PrefetchScalarGridSpec(
            num_scalar_prefetch=1,
            grid=(grid,),
            in_specs=[
                pl.BlockSpec((4, tile_b), lambda b, s: (0, b)),
                full(zp.shape),
                full(w1_a.shape),
                full(w2_f.shape), full(w3_f.shape), full(b3_f.shape),
            ],
            out_specs=pl.BlockSpec((3, tile_b), lambda b, s: (0, b)),
        ),
        compiler_params=pltpu.CompilerParams(
            dimension_semantics=("parallel",),
        ),
    )(seed_arr, x_t, zp, w1_a, w2_f, w3_f, b3_f)

    return out_t.T
